# baseline (speedup 1.0000x reference)
import functools

import jax
import jax.numpy as jnp
from jax import lax
from jax.experimental import pallas as pl
from jax.experimental.pallas import tpu as pltpu
from jax.experimental.pallas import tpu_sc as plsc


def _make_proj(n, d, c):
    def _proj_kernel(x_ref, w_ref, yt_ref, zt_ref):
        x = x_ref[...]
        w = w_ref[...]
        dn = (((0,), (1,)), ((), ()))
        yt_ref[...] = lax.dot_general(
            w[0:d], x, dn, preferred_element_type=jnp.float32
        )
        zt_ref[...] = (
            lax.dot_general(w[d : 2 * d], x, dn, preferred_element_type=jnp.float32)
            + w[2 * d : 2 * d + 1].T
        )

    return _proj_kernel


@functools.cache
def _make_segsum(n, e, c, nw, nc, epw):
    mesh = plsc.VectorSubcoreMesh(core_axis_name="c", subcore_axis_name="s")
    epw_pad = -(-(epw + 112) // 128) * 128

    @functools.partial(
        pl.kernel,
        mesh=mesh,
        compiler_params=pltpu.CompilerParams(needs_layout_passes=False),
        out_type=tuple(
            jax.ShapeDtypeStruct((nw, n), jnp.float32) for _ in range(c + 1)
        ),
        scratch_types=[
            pltpu.VMEM((2, epw_pad), jnp.int32),
            pltpu.VMEM((c, n), jnp.float32),
        ]
        + [pltpu.VMEM((n,), jnp.float32) for _ in range(c + 1)]
        + [pltpu.SemaphoreType.DMA, pltpu.SemaphoreType.DMA],
    )
    def seg(ei_hbm, y_hbm, *rest):
        out_hbm = rest[: c + 1]
        ei_v, y_v = rest[c + 1 : c + 3]
        acc_v = rest[c + 3 : c + 4 + c]
        sem_e, sem_y = rest[c + 4 + c :]

        wid = lax.axis_index("s") * nc + lax.axis_index("c")
        base = wid * epw
        ab = pl.multiple_of((base // 128) * 128, 128)
        off = base - ab
        h_e = pltpu.async_copy(
            ei_hbm.at[pl.ds(0, 2), pl.ds(ab, epw_pad)], ei_v, sem_e
        )
        h_y = pltpu.async_copy(y_hbm, y_v, sem_y)

        zf = jnp.zeros((16,), jnp.float32)

        for a in acc_v:

            @plsc.parallel_loop(0, n // 16, unroll=8)
            def _zero(i, _a=a):
                _a[pl.ds(i * 16, 16)] = zf

        h_e.wait()
        h_y.wait()

        ones = jnp.full((16,), 1.0, jnp.float32)
        rows = [jnp.full((16,), j, jnp.int32) for j in range(c)]

        @plsc.parallel_loop(0, epw // 16, unroll=16)
        def _body(i):
            b0 = pl.multiple_of(off + i * 16, 16)
            s = ei_v[0, pl.ds(b0, 16)]
            d = ei_v[1, pl.ds(b0, 16)]
            for j in range(c):
                yj = plsc.load_gather(y_v, [rows[j], s])
                plsc.addupdate_scatter(acc_v[j], [d], yj)
            plsc.addupdate_scatter(acc_v[c], [d], ones)

        hs = [
            pltpu.async_copy(a, o.at[wid], sem_e if k % 2 else sem_y)
            for k, (a, o) in enumerate(zip(acc_v, out_hbm))
        ]
        for h in hs:
            h.wait()

    return seg


def _make_final(n, c):
    def _final_kernel(ps0_ref, ps1_ref, pc_ref, zt_ref, ls_ref, out_ref):
        cnt = jnp.maximum(jnp.sum(pc_ref[...], axis=0, keepdims=True), 1.0)
        o0 = jnp.sum(ps0_ref[...], axis=0, keepdims=True) / cnt + zt_ref[0:1]
        o1 = jnp.sum(ps1_ref[...], axis=0, keepdims=True) / cnt + zt_ref[1:2]
        m = jnp.maximum(o0, o1)
        lse = m + jnp.log(jnp.exp(o0 - m) + jnp.exp(o1 - m))
        ls_ref[0:1] = o0 - lse
        ls_ref[1:2] = o1 - lse
        out_ref[0:1] = o0
        out_ref[1:2] = o1

    return _final_kernel


def kernel(x, edge_index, W_l, W_r, b):
    n, d = x.shape
    e = edge_index.shape[1]
    c = W_l.shape[1]

    wcat = jnp.concatenate([W_l, W_r, b[None, :]], axis=0)
    yt, zt = pl.pallas_call(
        _make_proj(n, d, c),
        out_shape=(
            jax.ShapeDtypeStruct((c, n), jnp.float32),
            jax.ShapeDtypeStruct((c, n), jnp.float32),
        ),
    )(x, wcat)

    nc, ns = 2, 16
    nw = nc * ns
    epw = e // nw
    seg = _make_segsum(n, e, c, nw, nc, epw)
    ps0, ps1, pcnt = seg(edge_index, yt)

    ls_t, out_t = pl.pallas_call(
        _make_final(n, c),
        out_shape=(
            jax.ShapeDtypeStruct((c, n), jnp.float32),
            jax.ShapeDtypeStruct((c, n), jnp.float32),
        ),
    )(ps0, ps1, pcnt, zt)

    return (ls_t.T, out_t.T)


# speedup vs baseline: 1.0398x; 1.0398x over previous
import functools

import jax
import jax.numpy as jnp
from jax import lax
from jax.experimental import pallas as pl
from jax.experimental.pallas import tpu as pltpu
from jax.experimental.pallas import tpu_sc as plsc


def _make_proj(n, d, c):
    def _proj_kernel(x_ref, wt_ref, b_ref, yt_ref, zt_ref):
        x = x_ref[...]
        wt = wt_ref[...]
        dn = (((1,), (1,)), ((), ()))
        yt_ref[...] = lax.dot_general(
            wt[0:c], x, dn, preferred_element_type=jnp.float32
        )
        zt_ref[...] = (
            lax.dot_general(wt[c : 2 * c], x, dn, preferred_element_type=jnp.float32)
            + b_ref[...].T
        )

    return _proj_kernel


@functools.cache
def _make_segsum(n, e, c, nw, nc, epw):
    mesh = plsc.VectorSubcoreMesh(core_axis_name="c", subcore_axis_name="s")
    epw_pad = -(-(epw + 112) // 128) * 128

    @functools.partial(
        pl.kernel,
        mesh=mesh,
        compiler_params=pltpu.CompilerParams(needs_layout_passes=False),
        out_type=tuple(
            jax.ShapeDtypeStruct((nw, n), jnp.float32) for _ in range(c + 1)
        ),
        scratch_types=[
            pltpu.VMEM((2, epw_pad), jnp.int32),
            pltpu.VMEM((c, n), jnp.float32),
        ]
        + [pltpu.VMEM((n,), jnp.float32) for _ in range(c + 1)]
        + [pltpu.SemaphoreType.DMA, pltpu.SemaphoreType.DMA],
    )
    def seg(ei_hbm, y_hbm, *rest):
        out_hbm = rest[: c + 1]
        ei_v, y_v = rest[c + 1 : c + 3]
        acc_v = rest[c + 3 : c + 4 + c]
        sem_e, sem_y = rest[c + 4 + c :]

        wid = lax.axis_index("s") * nc + lax.axis_index("c")
        base = wid * epw
        ab = pl.multiple_of((base // 128) * 128, 128)
        off = base - ab
        h_e = pltpu.async_copy(
            ei_hbm.at[pl.ds(0, 2), pl.ds(ab, epw_pad)], ei_v, sem_e
        )
        h_y = pltpu.async_copy(y_hbm, y_v, sem_y)

        zf = jnp.zeros((16,), jnp.float32)

        for a in acc_v:

            @plsc.parallel_loop(0, n // 16, unroll=8)
            def _zero(i, _a=a):
                _a[pl.ds(i * 16, 16)] = zf

        h_e.wait()
        h_y.wait()

        ones = jnp.full((16,), 1.0, jnp.float32)
        rows = [jnp.full((16,), j, jnp.int32) for j in range(c)]

        @plsc.parallel_loop(0, epw // 16, unroll=16)
        def _body(i):
            b0 = pl.multiple_of(off + i * 16, 16)
            s = ei_v[0, pl.ds(b0, 16)]
            d = ei_v[1, pl.ds(b0, 16)]
            for j in range(c):
                yj = plsc.load_gather(y_v, [rows[j], s])
                plsc.addupdate_scatter(acc_v[j], [d], yj)
            plsc.addupdate_scatter(acc_v[c], [d], ones)

        hs = [
            pltpu.async_copy(a, o.at[wid], sem_e if k % 2 else sem_y)
            for k, (a, o) in enumerate(zip(acc_v, out_hbm))
        ]
        for h in hs:
            h.wait()

    return seg


def _make_final(n, c):
    def _final_kernel(ps0_ref, ps1_ref, pc_ref, zt_ref, ls_ref, out_ref):
        cnt = jnp.maximum(jnp.sum(pc_ref[...], axis=0, keepdims=True), 1.0)
        o0 = jnp.sum(ps0_ref[...], axis=0, keepdims=True) / cnt + zt_ref[0:1]
        o1 = jnp.sum(ps1_ref[...], axis=0, keepdims=True) / cnt + zt_ref[1:2]
        m = jnp.maximum(o0, o1)
        lse = m + jnp.log(jnp.exp(o0 - m) + jnp.exp(o1 - m))
        ls_ref[0:1] = o0 - lse
        ls_ref[1:2] = o1 - lse
        out_ref[0:1] = o0
        out_ref[1:2] = o1

    return _final_kernel


def kernel(x, edge_index, W_l, W_r, b):
    n, d = x.shape
    e = edge_index.shape[1]
    c = W_l.shape[1]

    wcat_t = jnp.concatenate([W_l.T, W_r.T], axis=0)
    yt, zt = pl.pallas_call(
        _make_proj(n, d, c),
        out_shape=(
            jax.ShapeDtypeStruct((c, n), jnp.float32),
            jax.ShapeDtypeStruct((c, n), jnp.float32),
        ),
    )(x, wcat_t, b.reshape(1, c))

    nc, ns = 2, 16
    nw = nc * ns
    epw = e // nw
    seg = _make_segsum(n, e, c, nw, nc, epw)
    ps0, ps1, pcnt = seg(edge_index, yt)

    ls_t, out_t = pl.pallas_call(
        _make_final(n, c),
        out_shape=(
            jax.ShapeDtypeStruct((c, n), jnp.float32),
            jax.ShapeDtypeStruct((c, n), jnp.float32),
        ),
    )(ps0, ps1, pcnt, zt)

    return (ls_t.T, out_t.T)


# speedup vs baseline: 1.0932x; 1.0513x over previous
import functools

import jax
import jax.numpy as jnp
from jax import lax
from jax.experimental import pallas as pl
from jax.experimental.pallas import tpu as pltpu
from jax.experimental.pallas import tpu_sc as plsc


def _make_proj(n, d, c):
    def _proj_kernel(x_ref, wt_ref, b_ref, yp_ref, zt_ref):
        x = x_ref[...]
        wt = wt_ref[...]
        dn = (((1,), (1,)), ((), ()))
        yt = lax.dot_general(wt[0:c], x, dn, preferred_element_type=jnp.float32)
        u = lax.bitcast_convert_type(yt.astype(jnp.bfloat16), jnp.uint16)
        u32 = u.astype(jnp.uint32)
        yp_ref[...] = lax.bitcast_convert_type(
            (u32[0:1] << 16) | u32[1:2], jnp.int32
        )
        zt_ref[...] = (
            lax.dot_general(wt[c : 2 * c], x, dn, preferred_element_type=jnp.float32)
            + b_ref[...].T
        )

    return _proj_kernel


@functools.cache
def _make_segsum(n, e, c, nw, nc, epw):
    mesh = plsc.VectorSubcoreMesh(core_axis_name="c", subcore_axis_name="s")
    epw_pad = -(-(epw + 112) // 128) * 128

    @functools.partial(
        pl.kernel,
        mesh=mesh,
        compiler_params=pltpu.CompilerParams(needs_layout_passes=False),
        out_type=tuple(
            jax.ShapeDtypeStruct((nw, n), jnp.float32) for _ in range(c + 1)
        ),
        scratch_types=[
            pltpu.VMEM((2, epw_pad), jnp.int32),
            pltpu.VMEM((1, n), jnp.int32),
        ]
        + [pltpu.VMEM((n,), jnp.float32) for _ in range(c + 1)]
        + [pltpu.SemaphoreType.DMA, pltpu.SemaphoreType.DMA],
    )
    def seg(ei_hbm, y_hbm, *rest):
        out_hbm = rest[: c + 1]
        ei_v, y_v = rest[c + 1 : c + 3]
        acc_v = rest[c + 3 : c + 4 + c]
        sem_e, sem_y = rest[c + 4 + c :]

        wid = lax.axis_index("s") * nc + lax.axis_index("c")
        base = wid * epw
        ab = pl.multiple_of((base // 128) * 128, 128)
        off = base - ab
        h_e = pltpu.async_copy(
            ei_hbm.at[pl.ds(0, 2), pl.ds(ab, epw_pad)], ei_v, sem_e
        )
        h_y = pltpu.async_copy(y_hbm, y_v, sem_y)

        zf = jnp.zeros((16,), jnp.float32)

        for a in acc_v:

            @plsc.parallel_loop(0, n // 16, unroll=8)
            def _zero(i, _a=a):
                _a[pl.ds(i * 16, 16)] = zf

        h_e.wait()
        h_y.wait()

        ones = jnp.full((16,), 1.0, jnp.float32)
        row0 = jnp.zeros((16,), jnp.int32)
        himask = jnp.full((16,), -65536, jnp.int32)

        @plsc.parallel_loop(0, epw // 16, unroll=16)
        def _body(i):
            b0 = pl.multiple_of(off + i * 16, 16)
            s = ei_v[0, pl.ds(b0, 16)]
            d = ei_v[1, pl.ds(b0, 16)]
            p = plsc.load_gather(y_v, [row0, s])
            y0 = plsc.bitcast(p & himask, jnp.float32)
            y1 = plsc.bitcast(p << 16, jnp.float32)
            plsc.addupdate_scatter(acc_v[0], [d], y0)
            plsc.addupdate_scatter(acc_v[1], [d], y1)
            plsc.addupdate_scatter(acc_v[c], [d], ones)

        hs = [
            pltpu.async_copy(a, o.at[wid], sem_e if k % 2 else sem_y)
            for k, (a, o) in enumerate(zip(acc_v, out_hbm))
        ]
        for h in hs:
            h.wait()

    return seg


def _make_final(n, c):
    def _final_kernel(ps0_ref, ps1_ref, pc_ref, zt_ref, ls_ref, out_ref):
        cnt = jnp.maximum(jnp.sum(pc_ref[...], axis=0, keepdims=True), 1.0)
        o0 = jnp.sum(ps0_ref[...], axis=0, keepdims=True) / cnt + zt_ref[0:1]
        o1 = jnp.sum(ps1_ref[...], axis=0, keepdims=True) / cnt + zt_ref[1:2]
        m = jnp.maximum(o0, o1)
        lse = m + jnp.log(jnp.exp(o0 - m) + jnp.exp(o1 - m))
        ls_ref[0:1] = o0 - lse
        ls_ref[1:2] = o1 - lse
        out_ref[0:1] = o0
        out_ref[1:2] = o1

    return _final_kernel


def kernel(x, edge_index, W_l, W_r, b):
    n, d = x.shape
    e = edge_index.shape[1]
    c = W_l.shape[1]

    wcat_t = jnp.concatenate([W_l.T, W_r.T], axis=0)
    yt, zt = pl.pallas_call(
        _make_proj(n, d, c),
        out_shape=(
            jax.ShapeDtypeStruct((1, n), jnp.int32),
            jax.ShapeDtypeStruct((c, n), jnp.float32),
        ),
    )(x, wcat_t, b.reshape(1, c))

    nc, ns = 2, 16
    nw = nc * ns
    epw = e // nw
    seg = _make_segsum(n, e, c, nw, nc, epw)
    ps0, ps1, pcnt = seg(edge_index, yt)

    ls_t, out_t = pl.pallas_call(
        _make_final(n, c),
        out_shape=(
            jax.ShapeDtypeStruct((c, n), jnp.float32),
            jax.ShapeDtypeStruct((c, n), jnp.float32),
        ),
    )(ps0, ps1, pcnt, zt)

    return (ls_t.T, out_t.T)


# speedup vs baseline: 1.1138x; 1.0189x over previous
import functools

import jax
import jax.numpy as jnp
from jax import lax
from jax.experimental import pallas as pl
from jax.experimental.pallas import tpu as pltpu
from jax.experimental.pallas import tpu_sc as plsc


def _make_proj(n, d, c):
    def _proj_kernel(x_ref, wt_ref, b_ref, yp_ref, zt_ref):
        x = x_ref[...]
        wt = wt_ref[...]
        dn = (((1,), (1,)), ((), ()))
        r = lax.dot_general(wt, x, dn, preferred_element_type=jnp.float32)
        u = lax.bitcast_convert_type(r[0:c].astype(jnp.bfloat16), jnp.uint16)
        u32 = u.astype(jnp.uint32)
        yp_ref[...] = lax.bitcast_convert_type(
            (u32[0:1] << 16) | u32[1:2], jnp.int32
        )
        zt_ref[...] = r[c : 2 * c] + b_ref[...].T

    return _proj_kernel


@functools.cache
def _make_segsum(n, e, c, nw, nc, epw):
    mesh = plsc.VectorSubcoreMesh(core_axis_name="c", subcore_axis_name="s")
    epw_pad = -(-(epw + 112) // 128) * 128

    @functools.partial(
        pl.kernel,
        mesh=mesh,
        compiler_params=pltpu.CompilerParams(needs_layout_passes=False),
        out_type=tuple(
            jax.ShapeDtypeStruct((nw, n), jnp.float32) for _ in range(c + 1)
        ),
        scratch_types=[
            pltpu.VMEM((2, epw_pad), jnp.int32),
            pltpu.VMEM((1, n), jnp.int32),
        ]
        + [pltpu.VMEM((n,), jnp.float32) for _ in range(c + 1)]
        + [pltpu.SemaphoreType.DMA, pltpu.SemaphoreType.DMA],
    )
    def seg(ei_hbm, y_hbm, *rest):
        out_hbm = rest[: c + 1]
        ei_v, y_v = rest[c + 1 : c + 3]
        acc_v = rest[c + 3 : c + 4 + c]
        sem_e, sem_y = rest[c + 4 + c :]

        wid = lax.axis_index("s") * nc + lax.axis_index("c")
        base = wid * epw
        ab = pl.multiple_of((base // 128) * 128, 128)
        off = base - ab
        h_e = pltpu.async_copy(
            ei_hbm.at[pl.ds(0, 2), pl.ds(ab, epw_pad)], ei_v, sem_e
        )
        h_y = pltpu.async_copy(y_hbm, y_v, sem_y)

        zf = jnp.zeros((16,), jnp.float32)

        for a in acc_v:

            @plsc.parallel_loop(0, n // 16, unroll=8)
            def _zero(i, _a=a):
                _a[pl.ds(i * 16, 16)] = zf

        h_e.wait()
        h_y.wait()

        ones = jnp.full((16,), 1.0, jnp.float32)
        row0 = jnp.zeros((16,), jnp.int32)
        himask = jnp.full((16,), -65536, jnp.int32)

        @plsc.parallel_loop(0, epw // 16, unroll=16)
        def _body(i):
            b0 = pl.multiple_of(off + i * 16, 16)
            s = ei_v[0, pl.ds(b0, 16)]
            d = ei_v[1, pl.ds(b0, 16)]
            p = plsc.load_gather(y_v, [row0, s])
            y0 = plsc.bitcast(p & himask, jnp.float32)
            y1 = plsc.bitcast(p << 16, jnp.float32)
            plsc.addupdate_scatter(acc_v[0], [d], y0)
            plsc.addupdate_scatter(acc_v[1], [d], y1)
            plsc.addupdate_scatter(acc_v[c], [d], ones)

        hs = [
            pltpu.async_copy(a, o.at[wid], sem_e if k % 2 else sem_y)
            for k, (a, o) in enumerate(zip(acc_v, out_hbm))
        ]
        for h in hs:
            h.wait()

    return seg


def _make_final(n, c):
    def _final_kernel(ps0_ref, ps1_ref, pc_ref, zt_ref, ls_ref, out_ref):
        cnt = jnp.maximum(jnp.sum(pc_ref[...], axis=0, keepdims=True), 1.0)
        o0 = jnp.sum(ps0_ref[...], axis=0, keepdims=True) / cnt + zt_ref[0:1]
        o1 = jnp.sum(ps1_ref[...], axis=0, keepdims=True) / cnt + zt_ref[1:2]
        m = jnp.maximum(o0, o1)
        lse = m + jnp.log(jnp.exp(o0 - m) + jnp.exp(o1 - m))
        ls_ref[0:1] = o0 - lse
        ls_ref[1:2] = o1 - lse
        out_ref[0:1] = o0
        out_ref[1:2] = o1

    return _final_kernel


def kernel(x, edge_index, W_l, W_r, b):
    n, d = x.shape
    e = edge_index.shape[1]
    c = W_l.shape[1]

    wcat_t = jnp.concatenate([W_l.T, W_r.T], axis=0)
    yt, zt = pl.pallas_call(
        _make_proj(n, d, c),
        out_shape=(
            jax.ShapeDtypeStruct((1, n), jnp.int32),
            jax.ShapeDtypeStruct((c, n), jnp.float32),
        ),
    )(x, wcat_t, b.reshape(1, c))

    nc, ns = 2, 16
    nw = nc * ns
    epw = e // nw
    seg = _make_segsum(n, e, c, nw, nc, epw)
    ps0, ps1, pcnt = seg(edge_index, yt)

    ls_t, out_t = pl.pallas_call(
        _make_final(n, c),
        out_shape=(
            jax.ShapeDtypeStruct((c, n), jnp.float32),
            jax.ShapeDtypeStruct((c, n), jnp.float32),
        ),
    )(ps0, ps1, pcnt, zt)

    return (ls_t.T, out_t.T)


# speedup vs baseline: 1.1554x; 1.0373x over previous
import functools

import jax
import jax.numpy as jnp
from jax import lax
from jax.experimental import pallas as pl
from jax.experimental.pallas import tpu as pltpu
from jax.experimental.pallas import tpu_sc as plsc


def _make_proj(n, d, c):
    def _proj_kernel(x_ref, wlt_ref, wrt_ref, b_ref, yp_ref, zt_ref):
        x = x_ref[...]
        wt = jnp.concatenate([wlt_ref[...], wrt_ref[...]], axis=0)
        dn = (((1,), (1,)), ((), ()))
        r = lax.dot_general(wt, x, dn, preferred_element_type=jnp.float32)
        u = lax.bitcast_convert_type(r[0:c].astype(jnp.bfloat16), jnp.uint16)
        u32 = u.astype(jnp.uint32)
        yp_ref[...] = lax.bitcast_convert_type(
            (u32[0:1] << 16) | u32[1:2], jnp.int32
        )
        zt_ref[...] = r[c : 2 * c] + b_ref[...].T

    return _proj_kernel


@functools.cache
def _make_segsum(n, e, c, nw, nc, epw):
    mesh = plsc.VectorSubcoreMesh(core_axis_name="c", subcore_axis_name="s")
    epw_pad = -(-(epw + 112) // 128) * 128

    @functools.partial(
        pl.kernel,
        mesh=mesh,
        compiler_params=pltpu.CompilerParams(needs_layout_passes=False),
        out_type=tuple(
            jax.ShapeDtypeStruct((nw, n), jnp.float32) for _ in range(c + 1)
        ),
        scratch_types=[
            pltpu.VMEM((2, epw_pad), jnp.int32),
            pltpu.VMEM((1, n), jnp.int32),
        ]
        + [pltpu.VMEM((n,), jnp.float32) for _ in range(c + 1)]
        + [pltpu.SemaphoreType.DMA, pltpu.SemaphoreType.DMA],
    )
    def seg(ei_hbm, y_hbm, *rest):
        out_hbm = rest[: c + 1]
        ei_v, y_v = rest[c + 1 : c + 3]
        acc_v = rest[c + 3 : c + 4 + c]
        sem_e, sem_y = rest[c + 4 + c :]

        wid = lax.axis_index("s") * nc + lax.axis_index("c")
        base = wid * epw
        ab = pl.multiple_of((base // 128) * 128, 128)
        off = base - ab
        h_e = pltpu.async_copy(
            ei_hbm.at[pl.ds(0, 2), pl.ds(ab, epw_pad)], ei_v, sem_e
        )
        h_y = pltpu.async_copy(y_hbm, y_v, sem_y)

        zf = jnp.zeros((16,), jnp.float32)

        for a in acc_v:

            @plsc.parallel_loop(0, n // 16, unroll=8)
            def _zero(i, _a=a):
                _a[pl.ds(i * 16, 16)] = zf

        h_e.wait()
        h_y.wait()

        ones = jnp.full((16,), 1.0, jnp.float32)
        row0 = jnp.zeros((16,), jnp.int32)
        himask = jnp.full((16,), -65536, jnp.int32)

        @plsc.parallel_loop(0, epw // 16, unroll=16)
        def _body(i):
            b0 = pl.multiple_of(off + i * 16, 16)
            s = ei_v[0, pl.ds(b0, 16)]
            d = ei_v[1, pl.ds(b0, 16)]
            p = plsc.load_gather(y_v, [row0, s])
            y0 = plsc.bitcast(p & himask, jnp.float32)
            y1 = plsc.bitcast(p << 16, jnp.float32)
            plsc.addupdate_scatter(acc_v[0], [d], y0)
            plsc.addupdate_scatter(acc_v[1], [d], y1)
            plsc.addupdate_scatter(acc_v[c], [d], ones)

        hs = [
            pltpu.async_copy(a, o.at[wid], sem_e if k % 2 else sem_y)
            for k, (a, o) in enumerate(zip(acc_v, out_hbm))
        ]
        for h in hs:
            h.wait()

    return seg


def _make_final(n, c):
    def _final_kernel(ps0_ref, ps1_ref, pc_ref, zt_ref, ls_ref, out_ref):
        cnt = jnp.maximum(jnp.sum(pc_ref[...], axis=0, keepdims=True), 1.0)
        o0 = jnp.sum(ps0_ref[...], axis=0, keepdims=True) / cnt + zt_ref[0:1]
        o1 = jnp.sum(ps1_ref[...], axis=0, keepdims=True) / cnt + zt_ref[1:2]
        m = jnp.maximum(o0, o1)
        lse = m + jnp.log(jnp.exp(o0 - m) + jnp.exp(o1 - m))
        ls_ref[0:1] = o0 - lse
        ls_ref[1:2] = o1 - lse
        out_ref[0:1] = o0
        out_ref[1:2] = o1

    return _final_kernel


def kernel(x, edge_index, W_l, W_r, b):
    n, d = x.shape
    e = edge_index.shape[1]
    c = W_l.shape[1]

    yt, zt = pl.pallas_call(
        _make_proj(n, d, c),
        out_shape=(
            jax.ShapeDtypeStruct((1, n), jnp.int32),
            jax.ShapeDtypeStruct((c, n), jnp.float32),
        ),
    )(x, W_l.T, W_r.T, b.reshape(1, c))

    nc, ns = 2, 16
    nw = nc * ns
    epw = e // nw
    seg = _make_segsum(n, e, c, nw, nc, epw)
    ps0, ps1, pcnt = seg(edge_index, yt)

    ls_t, out_t = pl.pallas_call(
        _make_final(n, c),
        out_shape=(
            jax.ShapeDtypeStruct((c, n), jnp.float32),
            jax.ShapeDtypeStruct((c, n), jnp.float32),
        ),
    )(ps0, ps1, pcnt, zt)

    return (ls_t.T, out_t.T)


# speedup vs baseline: 1.1584x; 1.0026x over previous
import functools

import jax
import jax.numpy as jnp
from jax import lax
from jax.experimental import pallas as pl
from jax.experimental.pallas import tpu as pltpu
from jax.experimental.pallas import tpu_sc as plsc


def _make_proj(n, d, c):
    def _proj_kernel(x_ref, wlt_ref, wrt_ref, b_ref, yp_ref, zt_ref):
        x = x_ref[...]
        wt = jnp.concatenate([wlt_ref[...], wrt_ref[...]], axis=0)
        dn = (((1,), (1,)), ((), ()))
        r = lax.dot_general(wt, x, dn, preferred_element_type=jnp.float32)
        u = lax.bitcast_convert_type(r[0:c].astype(jnp.bfloat16), jnp.uint16)
        u32 = u.astype(jnp.uint32)
        yp_ref[...] = lax.bitcast_convert_type(
            (u32[0:1] << 16) | u32[1:2], jnp.int32
        )
        zt_ref[...] = r[c : 2 * c] + b_ref[...].T

    return _proj_kernel


@functools.cache
def _make_segsum(n, e, c, nw, nc, epw):
    mesh = plsc.VectorSubcoreMesh(core_axis_name="c", subcore_axis_name="s")
    epw_pad = -(-(epw + 112) // 128) * 128

    @functools.partial(
        pl.kernel,
        mesh=mesh,
        compiler_params=pltpu.CompilerParams(needs_layout_passes=False),
        out_type=tuple(
            jax.ShapeDtypeStruct((nw, n), jnp.float32) for _ in range(c + 1)
        ),
        scratch_types=[
            pltpu.VMEM((2, epw_pad), jnp.int32),
            pltpu.VMEM((1, n), jnp.int32),
        ]
        + [pltpu.VMEM((n,), jnp.float32) for _ in range(c + 1)]
        + [pltpu.SemaphoreType.DMA, pltpu.SemaphoreType.DMA],
    )
    def seg(ei_hbm, y_hbm, *rest):
        out_hbm = rest[: c + 1]
        ei_v, y_v = rest[c + 1 : c + 3]
        acc_v = rest[c + 3 : c + 4 + c]
        sem_e, sem_y = rest[c + 4 + c :]

        wid = lax.axis_index("s") * nc + lax.axis_index("c")
        base = wid * epw
        ab = pl.multiple_of((base // 128) * 128, 128)
        off = base - ab
        h_e = pltpu.async_copy(
            ei_hbm.at[pl.ds(0, 2), pl.ds(ab, epw_pad)], ei_v, sem_e
        )
        h_y = pltpu.async_copy(y_hbm, y_v, sem_y)

        zf = jnp.zeros((16,), jnp.float32)

        @plsc.parallel_loop(0, n // 16, unroll=4)
        def _zero(i):
            for a in acc_v:
                a[pl.ds(i * 16, 16)] = zf

        h_e.wait()
        h_y.wait()

        ones = jnp.full((16,), 1.0, jnp.float32)
        row0 = jnp.zeros((16,), jnp.int32)
        himask = jnp.full((16,), -65536, jnp.int32)

        @plsc.parallel_loop(0, epw // 16, unroll=8)
        def _body(i):
            b0 = pl.multiple_of(off + i * 16, 16)
            s = ei_v[0, pl.ds(b0, 16)]
            d = ei_v[1, pl.ds(b0, 16)]
            p = plsc.load_gather(y_v, [row0, s])
            y0 = plsc.bitcast(p & himask, jnp.float32)
            y1 = plsc.bitcast(p << 16, jnp.float32)
            plsc.addupdate_scatter(acc_v[0], [d], y0)
            plsc.addupdate_scatter(acc_v[1], [d], y1)
            plsc.addupdate_scatter(acc_v[c], [d], ones)

        hs = [
            pltpu.async_copy(a, o.at[wid], sem_e if k % 2 else sem_y)
            for k, (a, o) in enumerate(zip(acc_v, out_hbm))
        ]
        for h in hs:
            h.wait()

    return seg


def _make_final(n, c):
    def _final_kernel(ps0_ref, ps1_ref, pc_ref, zt_ref, ls_ref, out_ref):
        cnt = jnp.maximum(jnp.sum(pc_ref[...], axis=0, keepdims=True), 1.0)
        o0 = jnp.sum(ps0_ref[...], axis=0, keepdims=True) / cnt + zt_ref[0:1]
        o1 = jnp.sum(ps1_ref[...], axis=0, keepdims=True) / cnt + zt_ref[1:2]
        m = jnp.maximum(o0, o1)
        lse = m + jnp.log(jnp.exp(o0 - m) + jnp.exp(o1 - m))
        ls_ref[0:1] = o0 - lse
        ls_ref[1:2] = o1 - lse
        out_ref[0:1] = o0
        out_ref[1:2] = o1

    return _final_kernel


def kernel(x, edge_index, W_l, W_r, b):
    n, d = x.shape
    e = edge_index.shape[1]
    c = W_l.shape[1]

    yt, zt = pl.pallas_call(
        _make_proj(n, d, c),
        out_shape=(
            jax.ShapeDtypeStruct((1, n), jnp.int32),
            jax.ShapeDtypeStruct((c, n), jnp.float32),
        ),
    )(x, W_l.T, W_r.T, b.reshape(1, c))

    nc, ns = 2, 16
    nw = nc * ns
    epw = e // nw
    seg = _make_segsum(n, e, c, nw, nc, epw)
    ps0, ps1, pcnt = seg(edge_index, yt)

    ls_t, out_t = pl.pallas_call(
        _make_final(n, c),
        out_shape=(
            jax.ShapeDtypeStruct((c, n), jnp.float32),
            jax.ShapeDtypeStruct((c, n), jnp.float32),
        ),
    )(ps0, ps1, pcnt, zt)

    return (ls_t.T, out_t.T)
